# Initial kernel scaffold; baseline (speedup 1.0000x reference)
import jax
import jax.numpy as jnp
from jax.experimental import pallas as pl


def kernel(embed_node, edge_w1, edge_b1, edge_w2, edge_b2, x, edge_attr):
    raise NotImplementedError("write your pallas kernel here")


# speedup vs baseline: 2.4233x; 2.4233x over previous
import jax
import jax.numpy as jnp
from jax.experimental import pallas as pl
from jax.experimental.pallas import tpu as pltpu

LANE = 128
TILE = 4096
_VMEM_LIMIT = 64 * 1024 * 1024


def _round_up(a, m):
    return ((a + m - 1) // m) * m


def _pad2(a, rows, cols):
    return jnp.pad(a, ((0, rows - a.shape[0]), (0, cols - a.shape[1])))


def _fused_kernel(x_ref, e_ref, table_ref, w1_ref, b1_ref, w2_ref, b2_ref,
                  node_out_ref, edge_out_ref):
    x = x_ref[...]
    t = x.shape[0]
    s_pad = table_ref.shape[0]
    cols = jax.lax.broadcasted_iota(jnp.int32, (t, s_pad), 1)
    onehot = (x == cols).astype(jnp.float32)
    node_out_ref[...] = jnp.dot(
        onehot, table_ref[...], preferred_element_type=jnp.float32
    )

    e = e_ref[...]
    h = e * w1_ref[...] + b1_ref[...]
    h = h * jax.nn.sigmoid(h)
    out = jnp.dot(h, w2_ref[...], preferred_element_type=jnp.float32)
    edge_out_ref[...] = out + b2_ref[...]


def kernel(embed_node, edge_w1, edge_b1, edge_w2, edge_b2, x, edge_attr):
    n = x.shape[0]
    s, d = embed_node.shape
    e_rows = edge_attr.shape[0]
    assert e_rows == n, "fused kernel assumes n_nodes == n_edges"

    d_pad = _round_up(d, LANE)
    s_pad = _round_up(s, LANE)
    tile = min(TILE, _round_up(n, 8))
    n_pad = _round_up(n, tile)

    x2 = jnp.pad(x.astype(jnp.int32).reshape(n, 1), ((0, n_pad - n), (0, 0)))
    ea = jnp.pad(edge_attr, ((0, n_pad - n), (0, 0)))
    table_p = _pad2(embed_node, s_pad, d_pad)
    w1p = _pad2(edge_w1, 1, d_pad)
    b1p = _pad2(edge_b1, 1, d_pad)
    w2p = _pad2(edge_w2, d_pad, d_pad)
    b2p = _pad2(edge_b2, 1, d_pad)

    h_node, h_edge = pl.pallas_call(
        _fused_kernel,
        out_shape=(
            jax.ShapeDtypeStruct((n_pad, d_pad), jnp.float32),
            jax.ShapeDtypeStruct((n_pad, d_pad), jnp.float32),
        ),
        grid=(n_pad // tile,),
        in_specs=[
            pl.BlockSpec((tile, 1), lambda i: (i, 0)),
            pl.BlockSpec((tile, 1), lambda i: (i, 0)),
            pl.BlockSpec((s_pad, d_pad), lambda i: (0, 0)),
            pl.BlockSpec((1, d_pad), lambda i: (0, 0)),
            pl.BlockSpec((1, d_pad), lambda i: (0, 0)),
            pl.BlockSpec((d_pad, d_pad), lambda i: (0, 0)),
            pl.BlockSpec((1, d_pad), lambda i: (0, 0)),
        ],
        out_specs=(
            pl.BlockSpec((tile, d_pad), lambda i: (i, 0)),
            pl.BlockSpec((tile, d_pad), lambda i: (i, 0)),
        ),
        compiler_params=pltpu.CompilerParams(
            dimension_semantics=("parallel",),
            vmem_limit_bytes=_VMEM_LIMIT,
        ),
    )(x2, ea, table_p, w1p, b1p, w2p, b2p)
    return h_node[:n, :d], h_edge[:n, :d]


# speedup vs baseline: 2.4731x; 1.0205x over previous
import jax
import jax.numpy as jnp
from jax.experimental import pallas as pl
from jax.experimental.pallas import tpu as pltpu

LANE = 128
TILE = 8192
_VMEM_LIMIT = 64 * 1024 * 1024


def _round_up(a, m):
    return ((a + m - 1) // m) * m


def _pad2(a, rows, cols):
    return jnp.pad(a, ((0, rows - a.shape[0]), (0, cols - a.shape[1])))


def _fused_kernel(x_ref, e_ref, table_ref, w1_ref, b1_ref, w2_ref, b2_ref,
                  node_out_ref, edge_out_ref):
    x = x_ref[...]
    t = x.shape[0]
    s_pad = table_ref.shape[0]
    cols = jax.lax.broadcasted_iota(jnp.int32, (t, s_pad), 1)
    onehot = (x == cols).astype(jnp.float32)
    node_out_ref[...] = jnp.dot(
        onehot, table_ref[...], preferred_element_type=jnp.float32
    )

    e = e_ref[...]
    h = e * w1_ref[...] + b1_ref[...]
    h = h * jax.nn.sigmoid(h)
    out = jnp.dot(h, w2_ref[...], preferred_element_type=jnp.float32)
    edge_out_ref[...] = out + b2_ref[...]


def kernel(embed_node, edge_w1, edge_b1, edge_w2, edge_b2, x, edge_attr):
    n = x.shape[0]
    s, d = embed_node.shape
    e_rows = edge_attr.shape[0]
    assert e_rows == n, "fused kernel assumes n_nodes == n_edges"

    d_pad = _round_up(d, LANE)
    s_pad = _round_up(s, LANE)
    tile = min(TILE, _round_up(n, 8))
    n_pad = _round_up(n, tile)

    x2 = jnp.pad(x.astype(jnp.int32).reshape(n, 1), ((0, n_pad - n), (0, 0)))
    ea = jnp.pad(edge_attr, ((0, n_pad - n), (0, 0)))
    table_p = _pad2(embed_node, s_pad, d_pad)
    w1p = _pad2(edge_w1, 1, d_pad)
    b1p = _pad2(edge_b1, 1, d_pad)
    w2p = _pad2(edge_w2, d_pad, d_pad)
    b2p = _pad2(edge_b2, 1, d_pad)

    h_node, h_edge = pl.pallas_call(
        _fused_kernel,
        out_shape=(
            jax.ShapeDtypeStruct((n_pad, d_pad), jnp.float32),
            jax.ShapeDtypeStruct((n_pad, d_pad), jnp.float32),
        ),
        grid=(n_pad // tile,),
        in_specs=[
            pl.BlockSpec((tile, 1), lambda i: (i, 0)),
            pl.BlockSpec((tile, 1), lambda i: (i, 0)),
            pl.BlockSpec((s_pad, d_pad), lambda i: (0, 0)),
            pl.BlockSpec((1, d_pad), lambda i: (0, 0)),
            pl.BlockSpec((1, d_pad), lambda i: (0, 0)),
            pl.BlockSpec((d_pad, d_pad), lambda i: (0, 0)),
            pl.BlockSpec((1, d_pad), lambda i: (0, 0)),
        ],
        out_specs=(
            pl.BlockSpec((tile, d_pad), lambda i: (i, 0)),
            pl.BlockSpec((tile, d_pad), lambda i: (i, 0)),
        ),
        compiler_params=pltpu.CompilerParams(
            dimension_semantics=("parallel",),
            vmem_limit_bytes=_VMEM_LIMIT,
        ),
    )(x2, ea, table_p, w1p, b1p, w2p, b2p)
    return h_node[:n, :d], h_edge[:n, :d]
